# baseline (speedup 1.0000x reference)
import functools

import jax
import jax.numpy as jnp
from jax import lax
from jax.experimental import pallas as pl
from jax.experimental.pallas import tpu as pltpu
from jax.experimental.pallas import tpu_sc as plsc

_INFO = plsc.get_sparse_core_info()
_NC, _NS = _INFO.num_cores, _INFO.num_subcores

_FPG = 8
_HS = 4


def _make_gather(vocab: int, dim: int, hist: int, batch: int):
  per_sc = dim // _NC
  n_groups = -(-per_sc // _FPG)
  fpg_tail = per_sc - (n_groups - 1) * _FPG
  n_hg = hist // _HS
  item_elems = _HS * batch
  stage_big = -(-vocab // _NS) // 8 * 8
  stage_last = vocab - (_NS - 1) * stage_big
  assert hist % _HS == 0 and stage_last > 0 and stage_last % 8 == 0
  mesh = plsc.VectorSubcoreMesh(core_axis_name="c", subcore_axis_name="s")

  @functools.partial(
      pl.kernel,
      mesh=mesh,
      out_type=jax.ShapeDtypeStruct((dim, hist, batch), jnp.float32),
      scratch_types=[
          pltpu.VMEM_SHARED((_FPG * vocab,), jnp.float32),
          pltpu.VMEM((item_elems,), jnp.int32),
          pltpu.VMEM((item_elems,), jnp.float32),
          pltpu.VMEM((item_elems,), jnp.float32),
          pltpu.SemaphoreType.DMA,
          pltpu.SemaphoreType.DMA,
          pltpu.SemaphoreType.DMA,
      ],
  )
  def gather_kernel(tab_hbm, idx_hbm, out_hbm, vecs_sh, idx_v, buf_a, buf_b,
                    gsem, osem_a, osem_b):
    c = lax.axis_index("c")
    s = lax.axis_index("s")
    f_sc = c * per_sc

    def drain_item(buf, osem):
      for r in range(_HS):
        pltpu.make_async_copy(buf.at[pl.ds(r * batch, batch)],
                              out_hbm.at[0, 0], osem).wait()

    def group_body(g, carry):
      fpg_g = jnp.where(g == n_groups - 1, fpg_tail, _FPG)
      f_base = f_sc + g * _FPG

      def stage_one(fl, carry2):
        src0 = (f_base + fl) * vocab
        dst0 = fl * vocab

        @pl.when(s < _NS - 1)
        def _big():
          pltpu.sync_copy(tab_hbm.at[pl.ds(src0 + s * stage_big, stage_big)],
                          buf_a.at[pl.ds(0, stage_big)])
          pltpu.sync_copy(buf_a.at[pl.ds(0, stage_big)],
                          vecs_sh.at[pl.ds(dst0 + s * stage_big, stage_big)])

        @pl.when(s == _NS - 1)
        def _last():
          pltpu.sync_copy(
              tab_hbm.at[pl.ds(src0 + (_NS - 1) * stage_big, stage_last)],
              buf_a.at[pl.ds(0, stage_last)])
          pltpu.sync_copy(
              buf_a.at[pl.ds(0, stage_last)],
              vecs_sh.at[pl.ds(dst0 + (_NS - 1) * stage_big, stage_last)])

        return carry2

      lax.fori_loop(0, fpg_g, stage_one, 0)
      plsc.subcore_barrier()

      n_items = n_hg * fpg_g
      k0 = n_items * s // _NS
      k1 = n_items * (s + 1) // _NS
      n_k = k1 - k0

      def do_item(i, fl, hg, need_idx, buf, osem):
        @pl.when(i >= 2)
        def _drain():
          drain_item(buf, osem)

        @pl.when(need_idx)
        def _load_idx():
          pltpu.sync_copy(idx_hbm.at[pl.ds(hg * item_elems, item_elems)],
                          idx_v)

        pltpu.async_copy(
            vecs_sh.at[pl.ds(fl * vocab, vocab)].at[idx_v],
            buf,
            gsem,
        ).wait()
        for r in range(_HS):
          pltpu.async_copy(
              buf.at[pl.ds(r * batch, batch)],
              out_hbm.at[f_base + fl, hg * _HS + r],
              osem,
          )

      def item_body(i, state):
        prev_hg, fl, hg = state
        need_idx = hg != prev_hg

        @pl.when(lax.rem(i, 2) == 0)
        def _even():
          do_item(i, fl, hg, need_idx, buf_a, osem_a)

        @pl.when(lax.rem(i, 2) == 1)
        def _odd():
          do_item(i, fl, hg, need_idx, buf_b, osem_b)

        wrap = fl + 1 == fpg_g
        fl_n = jnp.where(wrap, 0, fl + 1)
        hg_n = jnp.where(wrap, hg + 1, hg)
        return hg, fl_n, hg_n

      fl0 = lax.rem(k0, fpg_g)
      hg0 = lax.div(k0, fpg_g)
      lax.fori_loop(0, n_k, item_body, (jnp.int32(-1), fl0, hg0))

      @pl.when(n_k >= 1)
      def _tail_a():
        drain_item(buf_a, osem_a)

      @pl.when(n_k >= 2)
      def _tail_b():
        drain_item(buf_b, osem_b)

      plsc.subcore_barrier()
      return carry

    lax.fori_loop(0, n_groups, group_body, 0)

  return gather_kernel


def kernel(x, table):
  batch, hist = x.shape
  vocab, dim = table.shape
  tab_lin = table.T.reshape(-1)
  x_lin = x.T.astype(jnp.int32).reshape(-1)
  out_t = _make_gather(vocab, dim, hist, batch)(tab_lin, x_lin)
  return jnp.transpose(out_t, (2, 1, 0))


# speedup vs baseline: 1.1554x; 1.1554x over previous
import functools

import jax
import jax.numpy as jnp
from jax import lax
from jax.experimental import pallas as pl
from jax.experimental.pallas import tpu as pltpu
from jax.experimental.pallas import tpu_sc as plsc

_INFO = plsc.get_sparse_core_info()
_NC, _NS = _INFO.num_cores, _INFO.num_subcores
_NW = _NC * _NS

_CHUNK = 128
_LANE = 128


def _make_gather(vocab: int, dim: int, dim_pad: int, n_idx: int):
  assert n_idx % (_NW * _CHUNK) == 0
  per_w = n_idx // _NW
  n_chunks = per_w // _CHUNK
  mesh = plsc.VectorSubcoreMesh(core_axis_name="c", subcore_axis_name="s")

  @functools.partial(
      pl.kernel,
      mesh=mesh,
      out_type=jax.ShapeDtypeStruct((n_idx, dim_pad), jnp.float32),
      scratch_types=[
          pltpu.VMEM((n_chunks, _CHUNK), jnp.int32),
          pltpu.VMEM((_CHUNK, dim_pad), jnp.float32),
          pltpu.SemaphoreType.DMA,
      ],
  )
  def gather_kernel(table_hbm, idx_hbm, out_hbm, idx_v, rows_v, sem):
    wid = lax.axis_index("s") * _NC + lax.axis_index("c")
    pltpu.sync_copy(idx_hbm.at[pl.ds(wid * n_chunks, n_chunks)], idx_v)
    base = wid * per_w

    def chunk_body(c, carry):
      pltpu.async_copy(table_hbm.at[idx_v.at[c]], rows_v, sem).wait()
      pltpu.sync_copy(rows_v, out_hbm.at[pl.ds(base + c * _CHUNK, _CHUNK)])
      return carry

    lax.fori_loop(0, n_chunks, chunk_body, 0)

  return gather_kernel


def kernel(x, table):
  batch, hist = x.shape
  vocab, dim = table.shape
  dim_pad = (dim + _LANE - 1) // _LANE * _LANE
  n_idx = batch * hist
  idx2d = x.reshape(n_idx // _CHUNK, _CHUNK).astype(jnp.int32)
  table_p = jnp.pad(table, ((0, 0), (0, dim_pad - dim)))
  out = _make_gather(vocab, dim, dim_pad, n_idx)(table_p, idx2d)
  return out[:, :dim].reshape(batch, hist, dim)


# speedup vs baseline: 1.2153x; 1.0518x over previous
import functools

import jax
import jax.numpy as jnp
from jax import lax
from jax.experimental import pallas as pl
from jax.experimental.pallas import tpu as pltpu
from jax.experimental.pallas import tpu_sc as plsc

_INFO = plsc.get_sparse_core_info()
_NC, _NS = _INFO.num_cores, _INFO.num_subcores
_NW = _NC * _NS

_CHUNK = 128
_LANE = 128


def _make_gather(vocab: int, dim: int, dim_pad: int, n_idx: int):
  assert n_idx % (_NW * _CHUNK) == 0
  per_w = n_idx // _NW
  n_chunks = per_w // _CHUNK
  assert n_chunks % 2 == 0
  mesh = plsc.VectorSubcoreMesh(core_axis_name="c", subcore_axis_name="s")

  @functools.partial(
      pl.kernel,
      mesh=mesh,
      out_type=jax.ShapeDtypeStruct((n_idx, dim_pad), jnp.float32),
      scratch_types=[
          pltpu.VMEM((n_chunks, _CHUNK), jnp.int32),
          pltpu.VMEM((_CHUNK, dim_pad), jnp.float32),
          pltpu.VMEM((_CHUNK, dim_pad), jnp.float32),
          pltpu.SemaphoreType.DMA,
          pltpu.SemaphoreType.DMA,
          pltpu.SemaphoreType.DMA,
          pltpu.SemaphoreType.DMA,
      ],
  )
  def gather_kernel(table_hbm, idx_hbm, out_hbm, idx_v, rows_a, rows_b,
                    gsem_a, gsem_b, osem_a, osem_b):
    wid = lax.axis_index("s") * _NC + lax.axis_index("c")
    pltpu.sync_copy(idx_hbm.at[pl.ds(wid * n_chunks, n_chunks)], idx_v)
    base = wid * per_w

    def fire_gather(c, rows, gsem):
      pltpu.async_copy(table_hbm.at[idx_v.at[c]], rows, gsem)

    def wait_gather(rows, gsem):
      pltpu.make_async_copy(table_hbm.at[pl.ds(0, _CHUNK)], rows, gsem).wait()

    def fire_out(c, rows, osem):
      pltpu.async_copy(rows, out_hbm.at[pl.ds(base + c * _CHUNK, _CHUNK)],
                       osem)

    def drain_out(rows, osem):
      pltpu.make_async_copy(rows, out_hbm.at[pl.ds(0, _CHUNK)], osem).wait()

    def chunk_body(i, carry):
      @pl.when(lax.rem(i, 2) == 0)
      def _even():
        @pl.when(i >= 2)
        def _():
          drain_out(rows_a, osem_a)

        fire_gather(i, rows_a, gsem_a)

        @pl.when(i >= 1)
        def _():
          wait_gather(rows_b, gsem_b)
          fire_out(i - 1, rows_b, osem_b)

      @pl.when(lax.rem(i, 2) == 1)
      def _odd():
        @pl.when(i >= 2)
        def _():
          drain_out(rows_b, osem_b)

        fire_gather(i, rows_b, gsem_b)
        wait_gather(rows_a, gsem_a)
        fire_out(i - 1, rows_a, osem_a)

      return carry

    lax.fori_loop(0, n_chunks, chunk_body, 0)

    wait_gather(rows_b, gsem_b)
    fire_out(n_chunks - 1, rows_b, osem_b)
    drain_out(rows_a, osem_a)
    drain_out(rows_b, osem_b)

  return gather_kernel


def kernel(x, table):
  batch, hist = x.shape
  vocab, dim = table.shape
  dim_pad = (dim + _LANE - 1) // _LANE * _LANE
  n_idx = batch * hist
  idx2d = x.reshape(n_idx // _CHUNK, _CHUNK).astype(jnp.int32)
  table_p = jnp.pad(table, ((0, 0), (0, dim_pad - dim)))
  out = _make_gather(vocab, dim, dim_pad, n_idx)(table_p, idx2d)
  return out[:, :dim].reshape(batch, hist, dim)
